# baseline (speedup 1.0000x reference)
import jax
import jax.numpy as jnp
from jax.experimental import pallas as pl
from jax.experimental.pallas import tpu as pltpu

N = 8192
BLK = 128
STEPS = N // BLK


def _fused_body(f_ref, w_ref, nx_ref, nf_ref, stage):
    i = pl.program_id(0)
    partial = jax.lax.dot_general(
        w_ref[...], f_ref[...],
        dimension_numbers=(((1,), (0,)), ((), ())),
        preferred_element_type=jnp.float32,
    )
    rows = BLK // 128
    stage[pl.ds(i * rows, rows), :] = partial.reshape(rows, 128)

    @pl.when(i == STEPS - 1)
    def _():
        v = stage[...]
        theta = jnp.mean(v)
        nx_ref[...] = v
        nf_ref[...] = (v > theta).astype(jnp.float32)


def kernel(x, is_firing, W):
    f2 = is_firing.reshape(N, 1)
    nx, nf = pl.pallas_call(
        _fused_body,
        grid=(STEPS,),
        in_specs=[
            pl.BlockSpec((N, 1), lambda i: (0, 0)),
            pl.BlockSpec((BLK, N), lambda i: (i, 0)),
        ],
        out_specs=(
            pl.BlockSpec((N // 128, 128), lambda i: (0, 0)),
            pl.BlockSpec((N // 128, 128), lambda i: (0, 0)),
        ),
        out_shape=(
            jax.ShapeDtypeStruct((N // 128, 128), jnp.float32),
            jax.ShapeDtypeStruct((N // 128, 128), jnp.float32),
        ),
        scratch_shapes=[pltpu.VMEM((N // 128, 128), jnp.float32)],

    )(f2, W)
    return nx.reshape(N), nf.reshape(N)


# speedup vs baseline: 1.2171x; 1.2171x over previous
import jax
import jax.numpy as jnp
from jax.experimental import pallas as pl
from jax.experimental.pallas import tpu as pltpu

N = 8192
BLK = 256
STEPS = N // BLK


def _fused_body(f_ref, w_ref, nx_ref, nf_ref, stage):
    i = pl.program_id(0)
    partial = jax.lax.dot_general(
        w_ref[...], f_ref[...],
        dimension_numbers=(((1,), (0,)), ((), ())),
        preferred_element_type=jnp.float32,
    )
    rows = BLK // 128
    stage[pl.ds(i * rows, rows), :] = partial.reshape(rows, 128)

    @pl.when(i == STEPS - 1)
    def _():
        v = stage[...]
        theta = jnp.mean(v)
        nx_ref[...] = v
        nf_ref[...] = (v > theta).astype(jnp.float32)


def kernel(x, is_firing, W):
    f2 = is_firing.reshape(N, 1)
    nx, nf = pl.pallas_call(
        _fused_body,
        grid=(STEPS,),
        in_specs=[
            pl.BlockSpec((N, 1), lambda i: (0, 0)),
            pl.BlockSpec((BLK, N), lambda i: (i, 0)),
        ],
        out_specs=(
            pl.BlockSpec((N // 128, 128), lambda i: (0, 0)),
            pl.BlockSpec((N // 128, 128), lambda i: (0, 0)),
        ),
        out_shape=(
            jax.ShapeDtypeStruct((N // 128, 128), jnp.float32),
            jax.ShapeDtypeStruct((N // 128, 128), jnp.float32),
        ),
        scratch_shapes=[pltpu.VMEM((N // 128, 128), jnp.float32)],

    )(f2, W)
    return nx.reshape(N), nf.reshape(N)
